# baseline (device time: 20436 ns/iter reference)
import jax
import jax.numpy as jnp
from jax import lax
from jax.experimental import pallas as pl
from jax.experimental.pallas import tpu as pltpu

N_DEV = 4
C_GLOBAL = 2048
EPS = 1e-5


def kernel(x, t_emb, W_scale, W_shift):
    b, s, c_loc = x.shape

    def body(x_hbm, t_ref, ws_ref, wsh_ref, out_hbm,
             xv_ref, ob_ref, own_ref, comm_ref,
             in_sems, out_sems, send_sems, recv_sems):
        my = lax.axis_index("i")

        in_dmas = []
        for k in range(b):
            dma = pltpu.make_async_copy(x_hbm.at[k], xv_ref.at[k], in_sems.at[k])
            dma.start()
            in_dmas.append(dma)

        barrier_sem = pltpu.get_barrier_semaphore()
        for d in (1, 2, 3):
            pl.semaphore_signal(
                barrier_sem, inc=1,
                device_id=((my + d) % N_DEV,),
                device_id_type=pl.DeviceIdType.MESH,
            )
        pl.semaphore_wait(barrier_sem, 3)

        for k in range(b):
            in_dmas[k].wait()
            xk = xv_ref[k].astype(jnp.bfloat16)
            own_ref[k, :] = jnp.sum(xk, axis=-1, dtype=jnp.float32)
            own_ref[b + k, :] = jnp.sum(xk * xk, axis=-1, dtype=jnp.float32)

        rdmas = []
        for d in (1, 2, 3):
            rdma = pltpu.make_async_remote_copy(
                src_ref=own_ref,
                dst_ref=comm_ref.at[3 - d],
                send_sem=send_sems.at[d - 1],
                recv_sem=recv_sems.at[3 - d],
                device_id=((my + d) % N_DEV,),
                device_id_type=pl.DeviceIdType.MESH,
            )
            rdma.start()
            rdmas.append(rdma)

        t = t_ref[...]
        scale = jnp.dot(t, ws_ref[...], preferred_element_type=jnp.float32)
        shift = jnp.dot(t, wsh_ref[...], preferred_element_type=jnp.float32)
        sc_b = (1.0 + scale).astype(jnp.bfloat16)
        sh_b = shift.astype(jnp.bfloat16)

        for rdma in rdmas:
            rdma.wait_recv()

        tot = (own_ref[...] + comm_ref[0] + comm_ref[1] + comm_ref[2])
        mean = tot[:b] / C_GLOBAL
        var = tot[b:] / C_GLOBAL - mean * mean
        inv = lax.rsqrt(var + EPS)
        mean_b = mean.astype(jnp.bfloat16)
        inv_b = inv.astype(jnp.bfloat16)

        out_dmas = []
        for k in range(b):
            xk = xv_ref[k].astype(jnp.bfloat16)
            mk = mean_b[k][:, None]
            ik = inv_b[k][:, None]
            ob_ref[k] = ((xk - mk) * ik) * sc_b[k][None, :] + sh_b[k][None, :]
            dma = pltpu.make_async_copy(ob_ref.at[k], out_hbm.at[k], out_sems.at[k])
            dma.start()
            out_dmas.append(dma)

        for dma in out_dmas:
            dma.wait()
        for rdma in rdmas:
            rdma.wait_send()

    return pl.pallas_call(
        body,
        out_shape=jax.ShapeDtypeStruct((b, s, c_loc), jnp.bfloat16),
        in_specs=[
            pl.BlockSpec(memory_space=pl.ANY),
            pl.BlockSpec(memory_space=pltpu.VMEM),
            pl.BlockSpec(memory_space=pltpu.VMEM),
            pl.BlockSpec(memory_space=pltpu.VMEM),
        ],
        out_specs=pl.BlockSpec(memory_space=pl.ANY),
        scratch_shapes=[
            pltpu.VMEM((b, s, c_loc), jnp.float32),
            pltpu.VMEM((b, s, c_loc), jnp.bfloat16),
            pltpu.VMEM((2 * b, s), jnp.float32),
            pltpu.VMEM((3, 2 * b, s), jnp.float32),
            pltpu.SemaphoreType.DMA((2,)),
            pltpu.SemaphoreType.DMA((2,)),
            pltpu.SemaphoreType.DMA((3,)),
            pltpu.SemaphoreType.DMA((3,)),
        ],
        compiler_params=pltpu.CompilerParams(collective_id=0),
    )(x, t_emb, W_scale, W_shift)


# device time: 17108 ns/iter; 1.1945x vs baseline; 1.1945x over previous
import jax
import jax.numpy as jnp
from jax import lax
from jax.experimental import pallas as pl
from jax.experimental.pallas import tpu as pltpu

N_DEV = 4
C_GLOBAL = 2048
EPS = 1e-5
NOUT = 4


def kernel(x, t_emb, W_scale, W_shift):
    b, s, c_loc = x.shape
    cs = s // NOUT

    def body(x_ref, t_ref, ws_ref, wsh_ref, out_hbm,
             ob_ref, own_ref, comm_ref, out_sems, send_sems, recv_sems):
        my = lax.axis_index("i")

        barrier_sem = pltpu.get_barrier_semaphore()
        for d in (1, 2, 3):
            pl.semaphore_signal(
                barrier_sem, inc=1,
                device_id=((my + d) % N_DEV,),
                device_id_type=pl.DeviceIdType.MESH,
            )
        pl.semaphore_wait(barrier_sem, 3)

        xb = x_ref[...].astype(jnp.bfloat16)
        psum = jnp.sum(xb, axis=-1, dtype=jnp.float32)
        psq = jnp.sum(xb * xb, axis=-1, dtype=jnp.float32)
        own_ref[...] = jnp.concatenate([psum, psq], axis=0)

        rdmas = []
        for d in (1, 2, 3):
            rdma = pltpu.make_async_remote_copy(
                src_ref=own_ref,
                dst_ref=comm_ref.at[3 - d],
                send_sem=send_sems.at[d - 1],
                recv_sem=recv_sems.at[3 - d],
                device_id=((my + d) % N_DEV,),
                device_id_type=pl.DeviceIdType.MESH,
            )
            rdma.start()
            rdmas.append(rdma)

        t = t_ref[...]
        scale = jnp.dot(t, ws_ref[...], preferred_element_type=jnp.float32)
        shift = jnp.dot(t, wsh_ref[...], preferred_element_type=jnp.float32)
        sc_b = (1.0 + scale).astype(jnp.bfloat16)[:, None, :]
        sh_b = shift.astype(jnp.bfloat16)[:, None, :]

        for rdma in rdmas:
            rdma.wait_recv()

        tot = (own_ref[...] + comm_ref[0] + comm_ref[1] + comm_ref[2])
        mean = tot[:b] / C_GLOBAL
        var = tot[b:] / C_GLOBAL - mean * mean
        inv = lax.rsqrt(var + EPS)
        mean_b = mean.astype(jnp.bfloat16)
        inv_b = inv.astype(jnp.bfloat16)

        out_dmas = []
        for k in range(NOUT):
            sl = pl.ds(k * cs, cs)
            xk = x_ref[:, sl, :].astype(jnp.bfloat16)
            mk = mean_b[:, k * cs:(k + 1) * cs, None]
            ik = inv_b[:, k * cs:(k + 1) * cs, None]
            ob_ref[:, sl, :] = ((xk - mk) * ik) * sc_b + sh_b
            dma = pltpu.make_async_copy(
                ob_ref.at[:, sl, :], out_hbm.at[:, sl, :], out_sems.at[k]
            )
            dma.start()
            out_dmas.append(dma)

        for dma in out_dmas:
            dma.wait()
        for rdma in rdmas:
            rdma.wait_send()

    return pl.pallas_call(
        body,
        out_shape=jax.ShapeDtypeStruct((b, s, c_loc), jnp.bfloat16),
        in_specs=[pl.BlockSpec(memory_space=pltpu.VMEM)] * 4,
        out_specs=pl.BlockSpec(memory_space=pl.ANY),
        scratch_shapes=[
            pltpu.VMEM((b, s, c_loc), jnp.bfloat16),
            pltpu.VMEM((2 * b, s), jnp.float32),
            pltpu.VMEM((3, 2 * b, s), jnp.float32),
            pltpu.SemaphoreType.DMA((NOUT,)),
            pltpu.SemaphoreType.DMA((3,)),
            pltpu.SemaphoreType.DMA((3,)),
        ],
        compiler_params=pltpu.CompilerParams(collective_id=0),
    )(x, t_emb, W_scale, W_shift)


# device time: 16335 ns/iter; 1.2511x vs baseline; 1.0473x over previous
import jax
import jax.numpy as jnp
from jax import lax
from jax.experimental import pallas as pl
from jax.experimental.pallas import tpu as pltpu

N_DEV = 4
C_GLOBAL = 2048
EPS = 1e-5


def kernel(x, t_emb, W_scale, W_shift):
    b, s, c_loc = x.shape

    def body(x_ref, t_ref, ws_ref, wsh_ref, out_ref,
             own_ref, comm_ref, send_sems, recv_sems):
        my = lax.axis_index("i")

        barrier_sem = pltpu.get_barrier_semaphore()
        for d in (1, 2, 3):
            pl.semaphore_signal(
                barrier_sem, inc=1,
                device_id=((my + d) % N_DEV,),
                device_id_type=pl.DeviceIdType.MESH,
            )
        pl.semaphore_wait(barrier_sem, 3)

        xb = x_ref[...].astype(jnp.bfloat16)
        psum = jnp.sum(xb, axis=-1, dtype=jnp.float32)
        psq = jnp.sum(xb * xb, axis=-1, dtype=jnp.float32)
        own_ref[...] = jnp.concatenate([psum, psq], axis=0)

        rdmas = []
        for d in (1, 2, 3):
            rdma = pltpu.make_async_remote_copy(
                src_ref=own_ref,
                dst_ref=comm_ref.at[3 - d],
                send_sem=send_sems.at[d - 1],
                recv_sem=recv_sems.at[3 - d],
                device_id=((my + d) % N_DEV,),
                device_id_type=pl.DeviceIdType.MESH,
            )
            rdma.start()
            rdmas.append(rdma)

        t = t_ref[...]
        scale = jnp.dot(t, ws_ref[...], preferred_element_type=jnp.float32)
        shift = jnp.dot(t, wsh_ref[...], preferred_element_type=jnp.float32)
        sc_b = (1.0 + scale).astype(jnp.bfloat16)[:, None, :]
        sh_b = shift.astype(jnp.bfloat16)[:, None, :]

        for rdma in rdmas:
            rdma.wait_recv()

        tot = (own_ref[...] + comm_ref[0] + comm_ref[1] + comm_ref[2])
        mean = tot[:b] / C_GLOBAL
        var = tot[b:] / C_GLOBAL - mean * mean
        inv = lax.rsqrt(var + EPS)
        mean_b = mean.astype(jnp.bfloat16)[:, :, None]
        inv_b = inv.astype(jnp.bfloat16)[:, :, None]
        out_ref[...] = ((xb - mean_b) * inv_b) * sc_b + sh_b

        for rdma in rdmas:
            rdma.wait_send()

    return pl.pallas_call(
        body,
        out_shape=jax.ShapeDtypeStruct((b, s, c_loc), jnp.bfloat16),
        in_specs=[pl.BlockSpec(memory_space=pltpu.VMEM)] * 4,
        out_specs=pl.BlockSpec(memory_space=pltpu.VMEM),
        scratch_shapes=[
            pltpu.VMEM((2 * b, s), jnp.float32),
            pltpu.VMEM((3, 2 * b, s), jnp.float32),
            pltpu.SemaphoreType.DMA((3,)),
            pltpu.SemaphoreType.DMA((3,)),
        ],
        compiler_params=pltpu.CompilerParams(collective_id=0),
    )(x, t_emb, W_scale, W_shift)


# device time: 15498 ns/iter; 1.3186x vs baseline; 1.0540x over previous
import jax
import jax.numpy as jnp
from jax import lax
from jax.experimental import pallas as pl
from jax.experimental.pallas import tpu as pltpu

N_DEV = 4
C_GLOBAL = 2048
EPS = 1e-5
NH = 2


def kernel(x, t_emb, W_scale, W_shift):
    b, s, c_loc = x.shape
    sh_len = s // NH

    def body(x_ref, t_ref, ws_ref, wsh_ref, out_ref,
             own_ref, comm_ref, send_sems, recv_sems):
        my = lax.axis_index("i")

        barrier_sem = pltpu.get_barrier_semaphore()
        for d in (1, 2, 3):
            pl.semaphore_signal(
                barrier_sem, inc=1,
                device_id=((my + d) % N_DEV,),
                device_id_type=pl.DeviceIdType.MESH,
            )
        pl.semaphore_wait(barrier_sem, 3)

        rdmas = []
        for h in range(NH):
            rows = slice(h * sh_len, (h + 1) * sh_len)
            xh = x_ref[:, rows, :].astype(jnp.bfloat16)
            psum = jnp.sum(xh, axis=-1, dtype=jnp.float32)
            psq = jnp.sum(xh * xh, axis=-1, dtype=jnp.float32)
            own_ref[h] = jnp.concatenate([psum, psq], axis=0)
            for d in (1, 2, 3):
                rdma = pltpu.make_async_remote_copy(
                    src_ref=own_ref.at[h],
                    dst_ref=comm_ref.at[h, 3 - d],
                    send_sem=send_sems.at[h, d - 1],
                    recv_sem=recv_sems.at[h, 3 - d],
                    device_id=((my + d) % N_DEV,),
                    device_id_type=pl.DeviceIdType.MESH,
                )
                rdma.start()
                rdmas.append(rdma)

        t = t_ref[...]
        scale = jnp.dot(t, ws_ref[...], preferred_element_type=jnp.float32)
        shift = jnp.dot(t, wsh_ref[...], preferred_element_type=jnp.float32)
        sc_b = (1.0 + scale).astype(jnp.bfloat16)[:, None, :]
        sh_b = shift.astype(jnp.bfloat16)[:, None, :]

        for h in range(NH):
            for i in range(3):
                rdmas[h * 3 + i].wait_recv()
            tot = (own_ref[h] + comm_ref[h, 0]
                   + comm_ref[h, 1] + comm_ref[h, 2])
            mean = tot[:b] / C_GLOBAL
            var = tot[b:] / C_GLOBAL - mean * mean
            inv = lax.rsqrt(var + EPS)
            mean_b = mean.astype(jnp.bfloat16)[:, :, None]
            inv_b = inv.astype(jnp.bfloat16)[:, :, None]
            rows = slice(h * sh_len, (h + 1) * sh_len)
            xh = x_ref[:, rows, :].astype(jnp.bfloat16)
            out_ref[:, rows, :] = ((xh - mean_b) * inv_b) * sc_b + sh_b

        for rdma in rdmas:
            rdma.wait_send()

    return pl.pallas_call(
        body,
        out_shape=jax.ShapeDtypeStruct((b, s, c_loc), jnp.bfloat16),
        in_specs=[pl.BlockSpec(memory_space=pltpu.VMEM)] * 4,
        out_specs=pl.BlockSpec(memory_space=pltpu.VMEM),
        scratch_shapes=[
            pltpu.VMEM((NH, 2 * b, sh_len), jnp.float32),
            pltpu.VMEM((NH, 3, 2 * b, sh_len), jnp.float32),
            pltpu.SemaphoreType.DMA((NH, 3)),
            pltpu.SemaphoreType.DMA((NH, 3)),
        ],
        compiler_params=pltpu.CompilerParams(collective_id=0),
    )(x, t_emb, W_scale, W_shift)


# device time: 15449 ns/iter; 1.3228x vs baseline; 1.0032x over previous
import jax
import jax.numpy as jnp
from jax import lax
from jax.experimental import pallas as pl
from jax.experimental.pallas import tpu as pltpu

N_DEV = 4
C_GLOBAL = 2048
EPS = 1e-5
NH = 4


def kernel(x, t_emb, W_scale, W_shift):
    b, s, c_loc = x.shape
    sh_len = s // NH

    def body(x_ref, t_ref, ws_ref, wsh_ref, out_ref,
             own_ref, comm_ref, send_sems, recv_sems):
        my = lax.axis_index("i")

        barrier_sem = pltpu.get_barrier_semaphore()
        for d in (1, 2, 3):
            pl.semaphore_signal(
                barrier_sem, inc=1,
                device_id=((my + d) % N_DEV,),
                device_id_type=pl.DeviceIdType.MESH,
            )
        pl.semaphore_wait(barrier_sem, 3)

        rdmas = []
        for h in range(NH):
            rows = slice(h * sh_len, (h + 1) * sh_len)
            xh = x_ref[:, rows, :].astype(jnp.bfloat16)
            psum = jnp.sum(xh, axis=-1, dtype=jnp.float32)
            psq = jnp.sum(xh * xh, axis=-1, dtype=jnp.float32)
            own_ref[h] = jnp.concatenate([psum, psq], axis=0)
            for d in (1, 2, 3):
                rdma = pltpu.make_async_remote_copy(
                    src_ref=own_ref.at[h],
                    dst_ref=comm_ref.at[h, 3 - d],
                    send_sem=send_sems.at[h, d - 1],
                    recv_sem=recv_sems.at[h, 3 - d],
                    device_id=((my + d) % N_DEV,),
                    device_id_type=pl.DeviceIdType.MESH,
                )
                rdma.start()
                rdmas.append(rdma)

        t = t_ref[...]
        scale = jnp.dot(t, ws_ref[...], preferred_element_type=jnp.float32)
        shift = jnp.dot(t, wsh_ref[...], preferred_element_type=jnp.float32)
        sc_b = (1.0 + scale).astype(jnp.bfloat16)[:, None, :]
        sh_b = shift.astype(jnp.bfloat16)[:, None, :]

        for h in range(NH):
            for i in range(3):
                rdmas[h * 3 + i].wait_recv()
            tot = (own_ref[h] + comm_ref[h, 0]
                   + comm_ref[h, 1] + comm_ref[h, 2])
            mean = tot[:b] / C_GLOBAL
            var = tot[b:] / C_GLOBAL - mean * mean
            inv = lax.rsqrt(var + EPS)
            mean_b = mean.astype(jnp.bfloat16)[:, :, None]
            inv_b = inv.astype(jnp.bfloat16)[:, :, None]
            rows = slice(h * sh_len, (h + 1) * sh_len)
            xh = x_ref[:, rows, :].astype(jnp.bfloat16)
            out_ref[:, rows, :] = ((xh - mean_b) * inv_b) * sc_b + sh_b

        for rdma in rdmas:
            rdma.wait_send()

    return pl.pallas_call(
        body,
        out_shape=jax.ShapeDtypeStruct((b, s, c_loc), jnp.bfloat16),
        in_specs=[pl.BlockSpec(memory_space=pltpu.VMEM)] * 4,
        out_specs=pl.BlockSpec(memory_space=pltpu.VMEM),
        scratch_shapes=[
            pltpu.VMEM((NH, 2 * b, sh_len), jnp.float32),
            pltpu.VMEM((NH, 3, 2 * b, sh_len), jnp.float32),
            pltpu.SemaphoreType.DMA((NH, 3)),
            pltpu.SemaphoreType.DMA((NH, 3)),
        ],
        compiler_params=pltpu.CompilerParams(collective_id=0),
    )(x, t_emb, W_scale, W_shift)
